# baseline (speedup 1.0000x reference)
import jax
from jax.experimental import pallas as pl
from jax.experimental.pallas import tpu as pltpu


_COLS = 2048
_CHUNK = 4088
_TOTAL = 16384
_NBUF = 2
_LAG = 1


def _copy_kernel(x_ref, o_ref, buf, in_sem, out_sem):
    offs = list(range(0, _TOTAL, _CHUNK))
    szs = [min(_CHUNK, _TOTAL - o) for o in offs]
    n = len(offs)

    def in_copy(i):
        b = i % _NBUF
        return pltpu.make_async_copy(
            x_ref.at[pl.ds(offs[i], szs[i])],
            buf.at[b, pl.ds(0, szs[i])],
            in_sem.at[b],
        )

    def out_copy(i):
        b = i % _NBUF
        return pltpu.make_async_copy(
            buf.at[b, pl.ds(0, szs[i])],
            o_ref.at[pl.ds(offs[i], szs[i])],
            out_sem.at[b],
        )

    for i in range(n):
        if i >= _NBUF:
            out_copy(i - _NBUF).wait()
        in_copy(i).start()
        if i >= _LAG:
            in_copy(i - _LAG).wait()
            out_copy(i - _LAG).start()
    for i in range(max(0, n - _LAG), n):
        in_copy(i).wait()
        out_copy(i).start()
    for i in range(max(0, n - _NBUF), n):
        out_copy(i).wait()


def kernel(x, s):
    total_rows = x.shape[0] * x.shape[1]
    x2 = x.reshape(total_rows, x.shape[2])
    out = pl.pallas_call(
        _copy_kernel,
        in_specs=[pl.BlockSpec(memory_space=pl.ANY)],
        out_specs=pl.BlockSpec(memory_space=pl.ANY),
        out_shape=jax.ShapeDtypeStruct(x2.shape, x.dtype),
        scratch_shapes=[
            pltpu.VMEM((_NBUF, _CHUNK, _COLS), x.dtype),
            pltpu.SemaphoreType.DMA((_NBUF,)),
            pltpu.SemaphoreType.DMA((_NBUF,)),
        ],
        compiler_params=pltpu.CompilerParams(
            vmem_limit_bytes=100 * 1024 * 1024,
        ),
    )(x2)
    return out.reshape(x.shape)


# speedup vs baseline: 1.0161x; 1.0161x over previous
import jax
from jax.experimental import pallas as pl
from jax.experimental.pallas import tpu as pltpu


_COLS = 2048
_TOTAL = 16384
_NBUF = 3
_LAG = 1
_SIZES = [1024, 2560, 2560, 2560, 2560, 2560, 2304, 256]
_MAXCH = max(_SIZES)


def _copy_kernel(x_ref, o_ref, buf, in_sem, out_sem):
    szs = _SIZES
    offs = [sum(szs[:i]) for i in range(len(szs))]
    n = len(offs)

    def in_copy(i):
        b = i % _NBUF
        return pltpu.make_async_copy(
            x_ref.at[pl.ds(offs[i], szs[i])],
            buf.at[b, pl.ds(0, szs[i])],
            in_sem.at[b],
        )

    def out_copy(i):
        b = i % _NBUF
        return pltpu.make_async_copy(
            buf.at[b, pl.ds(0, szs[i])],
            o_ref.at[pl.ds(offs[i], szs[i])],
            out_sem.at[b],
        )

    for i in range(n):
        if i >= _NBUF:
            out_copy(i - _NBUF).wait()
        in_copy(i).start()
        if i >= _LAG:
            in_copy(i - _LAG).wait()
            out_copy(i - _LAG).start()
    for i in range(max(0, n - _LAG), n):
        in_copy(i).wait()
        out_copy(i).start()
    for i in range(max(0, n - _NBUF), n):
        out_copy(i).wait()


def kernel(x, s):
    total_rows = x.shape[0] * x.shape[1]
    x2 = x.reshape(total_rows, x.shape[2])
    out = pl.pallas_call(
        _copy_kernel,
        in_specs=[pl.BlockSpec(memory_space=pl.ANY)],
        out_specs=pl.BlockSpec(memory_space=pl.ANY),
        out_shape=jax.ShapeDtypeStruct(x2.shape, x.dtype),
        scratch_shapes=[
            pltpu.VMEM((_NBUF, _MAXCH, _COLS), x.dtype),
            pltpu.SemaphoreType.DMA((_NBUF,)),
            pltpu.SemaphoreType.DMA((_NBUF,)),
        ],
        compiler_params=pltpu.CompilerParams(
            vmem_limit_bytes=100 * 1024 * 1024,
        ),
    )(x2)
    return out.reshape(x.shape)
